# baseline (speedup 1.0000x reference)
import jax
import jax.numpy as jnp
from jax.experimental import pallas as pl
from jax.experimental.pallas import tpu as pltpu

_N, _J = 1024, 50
_C = 1000
_BJ = 1


_BC = 200


def _onehot_block(tT_ref, out_ref, *, nc):
    tv = tT_ref[0]
    cbase = pl.program_id(1) * _BC
    cls = cbase + jax.lax.broadcasted_iota(jnp.int32, (_BJ, _BC, _N), 1)
    out_ref[...] = (cls == tv[:, None, :]).astype(jnp.float32)


def kernel(t, eye):
    del eye
    import functools
    tT = t.astype(jnp.int32).T.reshape(_J // _BJ, _BJ, _N)
    w = pl.pallas_call(
        functools.partial(_onehot_block, nc=_C // _BC),
        grid=(_J // _BJ, _C // _BC),
        in_specs=[pl.BlockSpec((1, _BJ, _N), lambda j, c: (j, 0, 0))],
        out_specs=pl.BlockSpec((_BJ, _BC, _N), lambda j, c: (j, c, 0)),
        out_shape=jax.ShapeDtypeStruct((_J, _C, _N), jnp.float32),
        compiler_params=pltpu.CompilerParams(
            dimension_semantics=("arbitrary", "arbitrary"),
            allow_input_fusion=[True],
        ),
    )(tT)
    return w.transpose(2, 1, 0)


# speedup vs baseline: 2.0017x; 2.0017x over previous
import jax
import jax.numpy as jnp
from jax.experimental import pallas as pl
from jax.experimental.pallas import tpu as pltpu

_N, _J = 1024, 50
_C = 1000
_BJ = 1


def _onehot_block(tT_ref, out_ref):
    tv = tT_ref[0]
    cls = jax.lax.broadcasted_iota(jnp.int32, (_BJ, _C, _N), 1)
    out_ref[...] = (cls == tv[:, None, :]).astype(jnp.float32)


def kernel(t, eye):
    del eye
    tT = t.astype(jnp.int32).T.reshape(_J // _BJ, _BJ, _N)
    w = pl.pallas_call(
        _onehot_block,
        grid=(_J // _BJ,),
        in_specs=[pl.BlockSpec((1, _BJ, _N), lambda j: (j, 0, 0))],
        out_specs=pl.BlockSpec((_BJ, _C, _N), lambda j: (j, 0, 0)),
        out_shape=jax.ShapeDtypeStruct((_J, _C, _N), jnp.float32),
        compiler_params=pltpu.CompilerParams(
            dimension_semantics=("arbitrary",),
            allow_input_fusion=[True],
        ),
    )(tT)
    return w.transpose(2, 1, 0)
